# Initial kernel scaffold; baseline (speedup 1.0000x reference)
import jax
import jax.numpy as jnp
from jax.experimental import pallas as pl


def kernel(idx, table):
    raise NotImplementedError("write your pallas kernel here")


# speedup vs baseline: 2.9780x; 2.9780x over previous
import functools

import jax
import jax.numpy as jnp
from jax import lax
from jax.experimental import pallas as pl
from jax.experimental.pallas import tpu as pltpu
from jax.experimental.pallas import tpu_sc as plsc


@functools.lru_cache(maxsize=None)
def _gather_fn(B, D, NC, NS, CH):
    NW = NC * NS
    b_per_w = B // NW
    n_ch = b_per_w // CH
    mesh = plsc.VectorSubcoreMesh(core_axis_name="c", subcore_axis_name="s")

    @functools.partial(
        pl.kernel,
        mesh=mesh,
        out_type=jax.ShapeDtypeStruct((B, D), jnp.float32),
        scratch_types=[
            pltpu.VMEM((n_ch, CH), jnp.int32),
            pltpu.VMEM((CH, D), jnp.float32),
            pltpu.SemaphoreType.DMA,
        ],
    )
    def k(table_hbm, idx_hbm, out_hbm, idx_v, rows_v, sem):
        wid = lax.axis_index("s") * NC + lax.axis_index("c")
        base = wid * b_per_w
        pltpu.sync_copy(idx_hbm.at[wid], idx_v)

        def body(j, carry):
            pltpu.async_copy(table_hbm.at[idx_v.at[j]], rows_v, sem).wait()
            pltpu.sync_copy(rows_v, out_hbm.at[pl.ds(base + j * CH, CH)])
            return carry

        lax.fori_loop(0, n_ch, body, 0)

    return k


def kernel(idx, table):
    B0, S = idx.shape
    V, D = table.shape
    B = B0 * S
    info = plsc.get_sparse_core_info()
    NC, NS = info.num_cores, info.num_subcores
    NW = NC * NS
    CH = 128
    idx_flat = idx.reshape(B).astype(jnp.int32).reshape(NW, B // (NW * CH), CH)
    out = _gather_fn(B, D, NC, NS, CH)(table, idx_flat)
    return out.reshape(B0, S, D)


# speedup vs baseline: 3.3530x; 1.1259x over previous
import functools

import jax
import jax.numpy as jnp
from jax import lax
from jax.experimental import pallas as pl
from jax.experimental.pallas import tpu as pltpu
from jax.experimental.pallas import tpu_sc as plsc


@functools.lru_cache(maxsize=None)
def _gather_fn(B, D, NC, NS, CH, NB):
    NW = NC * NS
    b_per_w = B // NW
    n_ch = b_per_w // CH
    n_grp = n_ch // NB
    mesh = plsc.VectorSubcoreMesh(core_axis_name="c", subcore_axis_name="s")

    @functools.partial(
        pl.kernel,
        mesh=mesh,
        out_type=jax.ShapeDtypeStruct((B, D), jnp.float32),
        scratch_types=[
            pltpu.VMEM((n_ch, CH), jnp.int32),
            pltpu.VMEM((NB, CH, D), jnp.float32),
        ]
        + [pltpu.SemaphoreType.DMA] * NB,
    )
    def k(table_hbm, idx_hbm, out_hbm, idx_v, rows_v, *gsems):
        wid = lax.axis_index("s") * NC + lax.axis_index("c")
        base = wid * b_per_w
        pltpu.sync_copy(idx_hbm.at[wid], idx_v)

        for b in range(NB):
            pltpu.async_copy(table_hbm.at[idx_v.at[b]], rows_v.at[b], gsems[b])

        def body(jo, carry):
            for b in range(NB):
                j = jo * NB + b
                pltpu.make_async_copy(
                    table_hbm.at[idx_v.at[j]], rows_v.at[b], gsems[b]
                ).wait()
                pltpu.sync_copy(rows_v.at[b], out_hbm.at[pl.ds(base + j * CH, CH)])

                @pl.when(jo < n_grp - 1)
                def _():
                    pltpu.async_copy(
                        table_hbm.at[idx_v.at[j + NB]], rows_v.at[b], gsems[b]
                    )

            return carry

        lax.fori_loop(0, n_grp, body, 0)

    return k


def kernel(idx, table):
    B0, S = idx.shape
    V, D = table.shape
    B = B0 * S
    info = plsc.get_sparse_core_info()
    NC, NS = info.num_cores, info.num_subcores
    NW = NC * NS
    CH = 128
    NB = 5
    idx_flat = idx.reshape(B).astype(jnp.int32).reshape(NW, B // (NW * CH), CH)
    out = _gather_fn(B, D, NC, NS, CH, NB)(table, idx_flat)
    return out.reshape(B0, S, D)
